# baseline (speedup 1.0000x reference)
import jax
import jax.numpy as jnp
from jax.experimental import pallas as pl
from jax.experimental.pallas import tpu as pltpu

_N = 8192
_D = 512
_K = 64
_ITERS = 10
_MAXF = 32
_CHUNK = 1024


def _selector_body(x_hbm, t_ref, sel_ref, f2t_ref,
                   xh_ref, xm_ref, xl_ref, x2_ref, c0_ref, chunk_ref, sem):
    t = t_ref[...]
    tn = t / jnp.clip(jnp.sqrt(jnp.sum(t * t)), 1e-8)
    tnb = tn.astype(jnp.bfloat16)

    cp0 = pltpu.make_async_copy(x_hbm.at[pl.ds(0, _K), :], c0_ref, sem)
    cp0.start()
    cp0.wait()

    def _fill(i, carry):
        cp = pltpu.make_async_copy(
            x_hbm.at[pl.ds(i * _CHUNK, _CHUNK), :], chunk_ref, sem)
        cp.start()
        cp.wait()
        rows = pl.ds(i * _CHUNK, _CHUNK)
        xc = chunk_ref[...]
        h = xc.astype(jnp.bfloat16)
        xh_ref[rows, :] = h
        r1 = xc - h.astype(jnp.float32)
        m = r1.astype(jnp.bfloat16)
        xm_ref[rows, :] = m
        xl_ref[rows, :] = (r1 - m.astype(jnp.float32)).astype(jnp.bfloat16)
        x2c = jnp.sum(xc * xc, axis=1, keepdims=True)
        x2_ref[rows, :] = x2c
        xnb = (xc / jnp.clip(jnp.sqrt(x2c), 1e-8)).astype(jnp.bfloat16)
        f2t_ref[rows] = jnp.dot(xnb, tnb.T,
                                preferred_element_type=jnp.float32)[:, 0]
        return carry

    jax.lax.fori_loop(0, _N // _CHUNK, _fill, 0)

    x2 = x2_ref[...]
    f2t = f2t_ref[...]

    kk = jax.lax.broadcasted_iota(jnp.int32, (1, _K), 1)

    def _labels(c):
        c2 = jnp.sum(c * c, axis=1)
        xc = jnp.dot(xh_ref[...], c.astype(jnp.bfloat16).T,
                     preferred_element_type=jnp.float32)
        d2 = x2 - 2.0 * xc + c2[None, :]
        return jnp.argmin(d2, axis=1).astype(jnp.int32)

    def _step(_, c):
        labels = _labels(c)
        ohb = (labels[:, None] == kk).astype(jnp.bfloat16)
        dims = (((0,), (0,)), ((), ()))
        mm = lambda b: jax.lax.dot_general(
            ohb, b, dims, preferred_element_type=jnp.float32)
        sums = mm(xh_ref[...]) + mm(xm_ref[...]) + mm(xl_ref[...])
        counts = jnp.sum(ohb.astype(jnp.float32), axis=0)
        return jnp.where(counts[:, None] > 0,
                         sums / jnp.clip(counts[:, None], 1.0, None), c)

    c = jax.lax.fori_loop(0, _ITERS, _step, c0_ref[...])
    labels = _labels(c)

    masked = jnp.where(labels[:, None] == kk, f2t[:, None], -1e9)
    top_score = jnp.max(masked, axis=0)
    n_iota = jax.lax.broadcasted_iota(jnp.int32, (_N, _K), 0)
    top_idx = jnp.min(
        jnp.where(masked == top_score[None, :], n_iota, _N), axis=0)

    s_col = top_score[:, None]
    s_row = top_score[None, :]
    i_iota = jax.lax.broadcasted_iota(jnp.int32, (_K, _K), 0)
    j_iota = jax.lax.broadcasted_iota(jnp.int32, (_K, _K), 1)
    before = (s_row > s_col) | ((s_row == s_col) & (j_iota < i_iota))
    rank = jnp.sum(before.astype(jnp.int32), axis=1)
    selected = (rank < _MAXF) & (top_score > -1e8)

    hit = (n_iota == top_idx[None, :]) & selected[None, :]
    sel_ref[...] = jnp.max(hit.astype(jnp.int32), axis=1)


@jax.jit
def _run(image_features, text_features):
    return pl.pallas_call(
        _selector_body,
        in_specs=[
            pl.BlockSpec(memory_space=pl.ANY),
            pl.BlockSpec(memory_space=pltpu.VMEM),
        ],
        out_shape=(
            jax.ShapeDtypeStruct((_N,), jnp.int32),
            jax.ShapeDtypeStruct((_N,), jnp.float32),
        ),
        scratch_shapes=[
            pltpu.VMEM((_N, _D), jnp.bfloat16),
            pltpu.VMEM((_N, _D), jnp.bfloat16),
            pltpu.VMEM((_N, _D), jnp.bfloat16),
            pltpu.VMEM((_N, 1), jnp.float32),
            pltpu.VMEM((_K, _D), jnp.float32),
            pltpu.VMEM((_CHUNK, _D), jnp.float32),
            pltpu.SemaphoreType.DMA,
        ],
    )(image_features, text_features)


def kernel(image_features, text_features):
    is_selected, f2t = _run(image_features, text_features)
    return is_selected, f2t, image_features


# speedup vs baseline: 1.0546x; 1.0546x over previous
import jax
import jax.numpy as jnp
from jax.experimental import pallas as pl

_N = 8192
_D = 512
_K = 64
_ITERS = 10
_MAXF = 32


def _selector_body(x_ref, t_ref, sel_ref, f2t_ref):
    x = x_ref[...]
    t = t_ref[...]

    x2 = jnp.sum(x * x, axis=1, keepdims=True)
    xn = x / jnp.clip(jnp.sqrt(x2), 1e-8)
    tn = t / jnp.clip(jnp.sqrt(jnp.sum(t * t)), 1e-8)
    f2t = jnp.dot(xn.astype(jnp.bfloat16), tn.astype(jnp.bfloat16).T,
                  preferred_element_type=jnp.float32)[:, 0]

    kk = jax.lax.broadcasted_iota(jnp.int32, (1, _K), 1)

    def _labels(c):
        c2 = jnp.sum(c * c, axis=1)
        d2 = x2 - 2.0 * jnp.dot(x, c.T) + c2[None, :]
        return jnp.argmin(d2, axis=1).astype(jnp.int32)

    def _step(_, c):
        labels = _labels(c)
        oh = (labels[:, None] == kk).astype(jnp.float32)
        sums = jax.lax.dot_general(
            oh, x, (((0,), (0,)), ((), ())),
            precision=jax.lax.Precision.HIGHEST)
        counts = jnp.sum(oh, axis=0)
        return jnp.where(counts[:, None] > 0,
                         sums / jnp.clip(counts[:, None], 1.0, None), c)

    c = jax.lax.fori_loop(0, _ITERS, _step, x[:_K, :])
    labels = _labels(c)

    masked = jnp.where(labels[:, None] == kk, f2t[:, None], -1e9)
    top_score = jnp.max(masked, axis=0)
    top_idx = jnp.argmax(masked, axis=0).astype(jnp.int32)

    s_col = top_score[:, None]
    s_row = top_score[None, :]
    i_iota = jax.lax.broadcasted_iota(jnp.int32, (_K, _K), 0)
    j_iota = jax.lax.broadcasted_iota(jnp.int32, (_K, _K), 1)
    before = (s_row > s_col) | ((s_row == s_col) & (j_iota < i_iota))
    rank = jnp.sum(before.astype(jnp.int32), axis=1)
    selected = (rank < _MAXF) & (top_score > -1e8)

    tid = jnp.where(selected, top_idx, _N)
    n_iota = jax.lax.broadcasted_iota(jnp.int32, (_N, _K), 0)
    hit = n_iota == tid[None, :]
    sel_ref[...] = jnp.max(hit.astype(jnp.int32), axis=1)
    f2t_ref[...] = f2t


@jax.jit
def _run(image_features, text_features):
    return pl.pallas_call(
        _selector_body,
        out_shape=(
            jax.ShapeDtypeStruct((_N,), jnp.int32),
            jax.ShapeDtypeStruct((_N,), jnp.float32),
        ),
    )(image_features, text_features)


def kernel(image_features, text_features):
    is_selected, f2t = _run(image_features, text_features)
    return is_selected, f2t, image_features
